# baseline (device time: 37419 ns/iter reference)
import jax
import jax.numpy as jnp
from jax import lax
from jax.experimental import pallas as pl
from jax.experimental.pallas import tpu as pltpu

Y = 4


def kernel(Q, K, V):
    b, sq, h, d = Q.shape
    scale = d ** -0.5

    def body(q_ref, k_ref, v_ref, out_ref, self_ref, comm_ref,
             send_sems, recv_sems):
        my_x = lax.axis_index("x")
        my_y = lax.axis_index("y")
        my_z = lax.axis_index("z")

        comm_ref[...] = jnp.zeros_like(comm_ref)

        barrier = pltpu.get_barrier_semaphore()
        for yy in range(Y):
            @pl.when(yy != my_y)
            def _():
                pl.semaphore_signal(
                    barrier, inc=1,
                    device_id=(my_x, yy, my_z),
                    device_id_type=pl.DeviceIdType.MESH,
                )
        pl.semaphore_wait(barrier, Y - 1)

        q = q_ref[...]
        k = k_ref[...]
        v = v_ref[...]
        s = jnp.sum(q * k, axis=-1, keepdims=True) * scale
        p = jnp.exp(s)
        l = jnp.sum(p, axis=1, keepdims=True)
        o = jnp.sum(p * v, axis=1)

        self_ref[:, 0:h, :] = o
        self_ref[:, h:h + 1, 0:h] = l[:, :, :, 0]

        for yy in range(Y):
            @pl.when(yy != my_y)
            def _():
                rdma = pltpu.make_async_remote_copy(
                    src_ref=self_ref,
                    dst_ref=comm_ref.at[my_y],
                    send_sem=send_sems.at[yy],
                    recv_sem=recv_sems.at[my_y],
                    device_id=(my_x, yy, my_z),
                    device_id_type=pl.DeviceIdType.MESH,
                )
                rdma.start()

        for src in range(Y):
            @pl.when(src != my_y)
            def _():
                rdma = pltpu.make_async_remote_copy(
                    src_ref=self_ref,
                    dst_ref=comm_ref.at[src],
                    send_sem=send_sems.at[src],
                    recv_sem=recv_sems.at[src],
                    device_id=(my_x, src, my_z),
                    device_id_type=pl.DeviceIdType.MESH,
                )
                rdma.wait_recv()

        total = self_ref[...]
        for src in range(Y):
            total = total + comm_ref[src]
        o_sum = total[:, 0:h, :]
        l_sum = total[:, h:h + 1, 0:h]
        l_bh = l_sum.reshape(b, h)[:, :, None]
        out_ref[:, 0, :, :] = o_sum / l_bh

        for yy in range(Y):
            @pl.when(yy != my_y)
            def _():
                rdma = pltpu.make_async_remote_copy(
                    src_ref=self_ref,
                    dst_ref=comm_ref.at[my_y],
                    send_sem=send_sems.at[yy],
                    recv_sem=recv_sems.at[my_y],
                    device_id=(my_x, yy, my_z),
                    device_id_type=pl.DeviceIdType.MESH,
                )
                rdma.wait_send()

    return pl.pallas_call(
        body,
        out_shape=jax.ShapeDtypeStruct((b, sq, h, d), jnp.float32),
        in_specs=[
            pl.BlockSpec(memory_space=pltpu.VMEM),
            pl.BlockSpec(memory_space=pltpu.VMEM),
            pl.BlockSpec(memory_space=pltpu.VMEM),
        ],
        out_specs=pl.BlockSpec(memory_space=pltpu.VMEM),
        scratch_shapes=[
            pltpu.VMEM((b, h + 1, d), jnp.float32),
            pltpu.VMEM((Y, b, h + 1, d), jnp.float32),
            pltpu.SemaphoreType.DMA((Y,)),
            pltpu.SemaphoreType.DMA((Y,)),
        ],
        compiler_params=pltpu.CompilerParams(collective_id=0),
    )(Q, K, V)
